# baseline (speedup 1.0000x reference)
import functools

import jax
import jax.numpy as jnp
from jax import lax
from jax.experimental import pallas as pl
from jax.experimental.pallas import tpu as pltpu


def _pack(w):
    co, ci, k, _ = w.shape
    return jnp.transpose(w, (0, 2, 3, 1)).reshape(co, k * k * ci)


def _fused_kernel(h, w, hw, hwp, k3, k5, c1, c3, c5, cr3, gb,
                  x_ref, wred_ref, w1_ref, w3_ref, w5_ref, wp_ref,
                  bred_ref, b1_ref, b3_ref, b5_ref, bp_ref, o_ref):
    f32 = jnp.float32
    bf16 = jnp.bfloat16
    i32 = jnp.int32

    pos = lax.broadcasted_iota(i32, (1, hwp), 1)
    yy = pos // w
    xx = pos - yy * w
    one = jnp.asarray(1.0, bf16)
    in_img_bf = jnp.where(pos < hw, f32(1.0), f32(0.0)).astype(bf16)

    def colmask(ox):
        m = (xx + ox >= 0) & (xx + ox < w)
        return jnp.where(m, f32(1.0), f32(0.0)).astype(bf16)

    def rowmask(oy):
        m = (yy + oy >= 0) & (yy + oy < h)
        return jnp.where(m, f32(1.0), f32(0.0)).astype(bf16)

    def shift_i(a, s):
        if s == 0:
            return a
        k = s % hwp
        return jnp.concatenate([a[:, k:], a[:, :k]], axis=-1)

    def shift_bf(a_bf, s):
        return pltpu.bitcast(shift_i(pltpu.bitcast(a_bf, i32), s), bf16)

    def conv(w_r, t, b_r):
        y = jnp.dot(w_r[...], t, preferred_element_type=f32)
        return jnp.maximum(y + b_r[...], 0.0)

    big = jnp.asarray(1e30, bf16)

    def one_image(j):
        xb = x_ref[j]

        yred = conv(wred_ref, xb, bred_ref)
        rb = yred.astype(bf16) * in_img_bf
        r3b = rb[:cr3]
        r5b = rb[cr3:]

        def taps(r, k):
            p = (k - 1) // 2
            cols = []
            for oy in range(-p, p + 1):
                for ox in range(-p, p + 1):
                    t = shift_bf(r, oy * w + ox)
                    if ox:
                        t = t * colmask(ox)
                    cols.append(t)
            return jnp.concatenate(cols, axis=0)

        y3 = conv(w3_ref, taps(r3b, k3), b3_ref)
        y5 = conv(w5_ref, taps(r5b, k5), b5_ref)
        y1 = conv(w1_ref, xb, b1_ref)

        def masked_shift(a, s, mf):
            return shift_bf(a, s) * mf + (mf - one) * big

        hmax = xb
        for ox in (-1, 1):
            hmax = jnp.maximum(hmax, masked_shift(xb, ox, colmask(ox)))
        pooled = hmax
        for oy in (-1, 1):
            pooled = jnp.maximum(pooled, masked_shift(hmax, oy * w, rowmask(oy)))
        y4 = conv(wp_ref, pooled, bp_ref)

        o_ref[j, 0:c1] = y1[:, :hw]
        o_ref[j, c1:c1 + c3] = y3[:, :hw]
        o_ref[j, c1 + c3:c1 + c3 + c5] = y5[:, :hw]
        o_ref[j, c1 + c3 + c5:] = y4[:, :hw]

    for j in range(gb):
        one_image(j)


@jax.jit
def kernel(x, w_b1, b_b1, w_b2a, b_b2a, w_b2b, b_b2b,
           w_b3a, b_b3a, w_b3b, b_b3b, w_b4, b_b4):
    n, cin, h, w = x.shape
    hw = h * w
    hwp = (hw + 127) // 128 * 128
    k3, k5 = w_b2b.shape[2], w_b3b.shape[2]
    c1, c3, c5, cp = w_b1.shape[0], w_b2b.shape[0], w_b3b.shape[0], w_b4.shape[0]
    cr3, cr5 = w_b2a.shape[0], w_b3a.shape[0]
    ct = c1 + c3 + c5 + cp

    bf16 = jnp.bfloat16
    xr = x.astype(bf16).reshape(n, cin, hw)
    x_ncm = jnp.pad(xr, ((0, 0), (0, 0), (0, hwp - hw)))

    wred = jnp.concatenate([_pack(w_b2a), _pack(w_b3a)], axis=0).astype(bf16)
    w1p = _pack(w_b1).astype(bf16)
    w3p = _pack(w_b2b).astype(bf16)
    w5p = _pack(w_b3b).astype(bf16)
    wpp = _pack(w_b4).astype(bf16)
    bred = jnp.concatenate([b_b2a, b_b3a]).reshape(-1, 1)
    b1r = b_b1.reshape(-1, 1)
    b3r = b_b2b.reshape(-1, 1)
    b5r = b_b3b.reshape(-1, 1)
    bpr = b_b4.reshape(-1, 1)

    gb = 2 if n % 2 == 0 else 1
    kern = functools.partial(_fused_kernel, h, w, hw, hwp, k3, k5,
                             c1, c3, c5, cr3, gb)
    img = lambda i: (i, 0, 0)
    rep = lambda i: (0, 0)
    out = pl.pallas_call(
        kern,
        out_shape=jax.ShapeDtypeStruct((n, ct, hw), jnp.float32),
        grid=(n // gb,),
        in_specs=[
            pl.BlockSpec((gb, cin, hwp), img),
            pl.BlockSpec((cr3 + cr5, cin), rep),
            pl.BlockSpec((c1, cin), rep),
            pl.BlockSpec((c3, k3 * k3 * cr3), rep),
            pl.BlockSpec((c5, k5 * k5 * cr5), rep),
            pl.BlockSpec((cp, cin), rep),
            pl.BlockSpec((cr3 + cr5, 1), rep),
            pl.BlockSpec((c1, 1), rep),
            pl.BlockSpec((c3, 1), rep),
            pl.BlockSpec((c5, 1), rep),
            pl.BlockSpec((cp, 1), rep),
        ],
        out_specs=pl.BlockSpec((gb, ct, hw), img),
        compiler_params=pltpu.CompilerParams(
            dimension_semantics=("parallel",),
            allow_input_fusion=[True] + [False] * 10,
            vmem_limit_bytes=32 * 1024 * 1024),
    )(x_ncm, wred, w1p, w3p, w5p, wpp, bred, b1r, b3r, b5r, bpr)
    return out.reshape(n, ct, h, w)


# speedup vs baseline: 1.1592x; 1.1592x over previous
import functools

import jax
import jax.numpy as jnp
from jax.experimental import pallas as pl
from jax.experimental.pallas import tpu as pltpu


def _pack(w):
    co, ci, k, _ = w.shape
    return jnp.transpose(w, (0, 2, 3, 1)).reshape(co, k * k * ci)


def _nc(taps, w_t, b, pref=jnp.float32):
    y = jnp.einsum('kn,kc->nc', taps, w_t, preferred_element_type=pref)
    return jnp.maximum(y + b[...], 0.0)


def _kernel_v2(h, w, k3, k5, c1, c3, c5, cp, cr3, cr5, cin, n,
               x_ref, wredt_ref, w1t_ref, w3t_ref, w5t_ref, wpt_ref,
               bredc_ref, b1_ref, b3_ref, b5_ref, bp_ref, o_ref,
               xring, yring):
    f32 = jnp.float32
    bf16 = jnp.bfloat16
    crt = cr3 + cr5
    t = pl.program_id(0)

    @pl.when(t == 0)
    def _init():
        xring[3] = jnp.zeros((w, cin, n), bf16)
        yring[6] = jnp.zeros((w, crt, n), bf16)
        yring[7] = jnp.zeros((w, crt, n), bf16)

    @pl.when(t < h)
    def _stage1():
        xring[t & 3] = x_ref[0]
        for col in range(w):
            xp = x_ref[0, col]
            yr = jnp.einsum('ck,cn->kn', wredt_ref[...], xp,
                            preferred_element_type=f32)
            yr = jnp.maximum(yr + bredc_ref[...], 0.0)
            yring[t & 7, col] = yr.astype(bf16)

    @pl.when(t >= h)
    def _pad_rows():
        yring[t & 7] = jnp.zeros((w, crt, n), bf16)

    @pl.when(t >= 2)
    def _emit_row():
        r = t - 2
        z3 = jnp.zeros((cr3, n), bf16)
        z5 = jnp.zeros((cr5, n), bf16)

        rm_up = jnp.where(r >= 1, f32(1.0), f32(0.0)).astype(bf16)
        rm_dn = jnp.where(r <= h - 2, f32(1.0), f32(0.0)).astype(bf16)
        neg_up = jnp.where(r >= 1, f32(0.0), f32(-1e30)).astype(bf16)
        neg_dn = jnp.where(r <= h - 2, f32(0.0), f32(-1e30)).astype(bf16)

        hrows = []
        for d in (-3, -2, -1):
            xs = xring.at[(t + d) & 3]
            loads = [xs[col] for col in range(w)]
            hr = []
            for col in range(w):
                m = loads[col]
                if col > 0:
                    m = jnp.maximum(m, loads[col - 1])
                if col < w - 1:
                    m = jnp.maximum(m, loads[col + 1])
                hr.append(m)
            hrows.append(hr)
        hup, hmid, hdn = hrows

        yup = yring.at[(t - 3) & 7]
        ymid = yring.at[(t - 2) & 7]
        ydn = yring.at[(t - 1) & 7]
        y3rows = (yup, ymid, ydn)
        y5rows = tuple(yring.at[(t - 4 + i) & 7] for i in range(k5))

        for col in range(w):
            t3 = []
            for rw in y3rows:
                for dx in range(-(k3 // 2), k3 // 2 + 1):
                    cc = col + dx
                    t3.append(rw[cc, 0:cr3] if 0 <= cc < w else z3)
            t5 = []
            for rw in y5rows:
                for dx in range(-(k5 // 2), k5 // 2 + 1):
                    cc = col + dx
                    t5.append(rw[cc, cr3:crt] if 0 <= cc < w else z5)
            y3 = _nc(jnp.concatenate(t3, axis=0), w3t_ref[...], b3_ref)
            y5 = _nc(jnp.concatenate(t5, axis=0), w5t_ref[...], b5_ref)

            xp = xring[(t - 2) & 3, col]
            y1 = _nc(xp, w1t_ref[...], b1_ref)

            pooled = jnp.maximum(
                hmid[col],
                jnp.maximum(hup[col] * rm_up + neg_up,
                            hdn[col] * rm_dn + neg_dn))
            y4 = _nc(pooled, wpt_ref[...], bp_ref)

            o_ref[0, col] = jnp.concatenate([y1, y3, y5, y4], axis=-1)


@jax.jit
def kernel(x, w_b1, b_b1, w_b2a, b_b2a, w_b2b, b_b2b,
           w_b3a, b_b3a, w_b3b, b_b3b, w_b4, b_b4):
    n, cin, h, w = x.shape
    k3, k5 = w_b2b.shape[2], w_b3b.shape[2]
    c1, c3, c5, cp = w_b1.shape[0], w_b2b.shape[0], w_b3b.shape[0], w_b4.shape[0]
    cr3, cr5 = w_b2a.shape[0], w_b3a.shape[0]
    ct = c1 + c3 + c5 + cp

    bf16 = jnp.bfloat16
    xt = jnp.transpose(x, (2, 3, 1, 0)).astype(bf16)

    wredt = jnp.concatenate([_pack(w_b2a), _pack(w_b3a)], axis=0).T.astype(bf16)
    w1t = _pack(w_b1).T.astype(bf16)
    w3t = _pack(w_b2b).T.astype(bf16)
    w5t = _pack(w_b3b).T.astype(bf16)
    wpt = _pack(w_b4).T.astype(bf16)
    bredc = jnp.concatenate([b_b2a, b_b3a]).reshape(-1, 1)
    b1r = b_b1.reshape(1, -1)
    b3r = b_b2b.reshape(1, -1)
    b5r = b_b3b.reshape(1, -1)
    bpr = b_b4.reshape(1, -1)

    kern = functools.partial(_kernel_v2, h, w, k3, k5,
                             c1, c3, c5, cp, cr3, cr5, cin, n)
    rowi = lambda t: (jnp.minimum(t, h - 1), 0, 0, 0)
    rowo = lambda t: (jnp.maximum(t - 2, 0), 0, 0, 0)
    rep2 = lambda t: (0, 0)
    out = pl.pallas_call(
        kern,
        out_shape=jax.ShapeDtypeStruct((h, w, n, ct), jnp.float32),
        grid=(h + 2,),
        in_specs=[
            pl.BlockSpec((1, w, cin, n), rowi),
            pl.BlockSpec((cin, cr3 + cr5), rep2),
            pl.BlockSpec((cin, c1), rep2),
            pl.BlockSpec((k3 * k3 * cr3, c3), rep2),
            pl.BlockSpec((k5 * k5 * cr5, c5), rep2),
            pl.BlockSpec((cin, cp), rep2),
            pl.BlockSpec((cr3 + cr5, 1), rep2),
            pl.BlockSpec((1, c1), rep2),
            pl.BlockSpec((1, c3), rep2),
            pl.BlockSpec((1, c5), rep2),
            pl.BlockSpec((1, cp), rep2),
        ],
        out_specs=pl.BlockSpec((1, w, n, ct), rowo),
        scratch_shapes=[
            pltpu.VMEM((4, w, cin, n), bf16),
            pltpu.VMEM((8, w, cr3 + cr5, n), bf16),
        ],
        compiler_params=pltpu.CompilerParams(
            dimension_semantics=("arbitrary",),
            fuse_transposed_lhs_in_matmul=True,
            vmem_limit_bytes=48 * 1024 * 1024),
    )(xt, wredt, w1t, w3t, w5t, wpt, bredc, b1r, b3r, b5r, bpr)
    return jnp.transpose(out, (2, 3, 0, 1))


# speedup vs baseline: 1.4728x; 1.2705x over previous
import functools

import jax
import jax.numpy as jnp
from jax.experimental import pallas as pl
from jax.experimental.pallas import tpu as pltpu


def _pack(w):
    co, ci, k, _ = w.shape
    return jnp.transpose(w, (0, 2, 3, 1)).reshape(co, k * k * ci)


def _nc(taps, w_t, b, pref=jnp.float32):
    y = jnp.einsum('kn,kc->nc', taps, w_t, preferred_element_type=pref)
    return jnp.maximum(y + b[...], 0.0)


def _kernel_v2(h, w, k3, k5, c1, c3, c5, cp, cr3, cr5, cin, n,
               x_ref, wredt_ref, wbig_ref, bbig_ref, bredc_ref, o_ref,
               xring, yring):
    f32 = jnp.float32
    bf16 = jnp.bfloat16
    crt = cr3 + cr5
    t = pl.program_id(0)

    @pl.when(t == 0)
    def _init():
        xring[3] = jnp.zeros((w, cin, n), bf16)
        yring[6] = jnp.zeros((w, crt, n), bf16)
        yring[7] = jnp.zeros((w, crt, n), bf16)

    px = 2 if w % 2 == 0 else 1

    @pl.when(t < h)
    def _stage1():
        xring[t & 3] = x_ref[0]
        for col in range(0, w, px):
            xp = jnp.concatenate([x_ref[0, col + j] for j in range(px)],
                                 axis=-1)
            yr = jnp.einsum('ck,cn->kn', wredt_ref[...], xp,
                            preferred_element_type=f32)
            yr = jnp.maximum(yr + bredc_ref[...], 0.0).astype(bf16)
            for j in range(px):
                yring[t & 7, col + j] = yr[:, j * n:(j + 1) * n]

    @pl.when(t >= h)
    def _pad_rows():
        yring[t & 7] = jnp.zeros((w, crt, n), bf16)

    @pl.when(t >= 2)
    def _emit_row():
        r = t - 2
        z3 = jnp.zeros((cr3, n), bf16)
        z5 = jnp.zeros((cr5, n), bf16)

        rm_up = jnp.where(r >= 1, f32(1.0), f32(0.0)).astype(bf16)
        rm_dn = jnp.where(r <= h - 2, f32(1.0), f32(0.0)).astype(bf16)
        neg_up = jnp.where(r >= 1, f32(0.0), f32(-1e30)).astype(bf16)
        neg_dn = jnp.where(r <= h - 2, f32(0.0), f32(-1e30)).astype(bf16)

        hrows = []
        for d in (-3, -2, -1):
            xs = xring.at[(t + d) & 3]
            loads = [xs[col] for col in range(w)]
            hr = []
            for col in range(w):
                m = loads[col]
                if col > 0:
                    m = jnp.maximum(m, loads[col - 1])
                if col < w - 1:
                    m = jnp.maximum(m, loads[col + 1])
                hr.append(m)
            hrows.append(hr)
        hup, hmid, hdn = hrows

        yup = yring.at[(t - 3) & 7]
        ymid = yring.at[(t - 2) & 7]
        ydn = yring.at[(t - 1) & 7]
        y3rows = (yup, ymid, ydn)
        y5rows = tuple(yring.at[(t - 4 + i) & 7] for i in range(k5))

        def lhs_for(col):
            t3 = []
            for rw in y3rows:
                for dx in range(-(k3 // 2), k3 // 2 + 1):
                    cc = col + dx
                    t3.append(rw[cc, 0:cr3] if 0 <= cc < w else z3)
            t5 = []
            for rw in y5rows:
                for dx in range(-(k5 // 2), k5 // 2 + 1):
                    cc = col + dx
                    t5.append(rw[cc, cr3:crt] if 0 <= cc < w else z5)
            pooled = jnp.maximum(
                hmid[col],
                jnp.maximum(hup[col] * rm_up + neg_up,
                            hdn[col] * rm_dn + neg_dn))
            xp = xring[(t - 2) & 3, col]
            return jnp.concatenate([xp] + t3 + t5 + [pooled], axis=0)

        for col in range(0, w, px):
            lhs = jnp.concatenate([lhs_for(col + j) for j in range(px)],
                                  axis=-1)
            y = _nc(lhs, wbig_ref[...], bbig_ref)
            for j in range(px):
                o_ref[0, col + j] = y[j * n:(j + 1) * n]


@jax.jit
def kernel(x, w_b1, b_b1, w_b2a, b_b2a, w_b2b, b_b2b,
           w_b3a, b_b3a, w_b3b, b_b3b, w_b4, b_b4):
    n, cin, h, w = x.shape
    k3, k5 = w_b2b.shape[2], w_b3b.shape[2]
    c1, c3, c5, cp = w_b1.shape[0], w_b2b.shape[0], w_b3b.shape[0], w_b4.shape[0]
    cr3, cr5 = w_b2a.shape[0], w_b3a.shape[0]
    ct = c1 + c3 + c5 + cp

    bf16 = jnp.bfloat16
    xt = jnp.transpose(x, (2, 3, 1, 0)).astype(bf16)

    wredt = jnp.concatenate([_pack(w_b2a), _pack(w_b3a)], axis=0).T.astype(bf16)
    bredc = jnp.concatenate([b_b2a, b_b3a]).reshape(-1, 1)

    k1r, k3r, k5r, kpr = cin, k3 * k3 * cr3, k5 * k5 * cr5, cin
    kt = k1r + k3r + k5r + kpr
    z = jnp.zeros
    f32 = jnp.float32
    col1 = jnp.concatenate([_pack(w_b1).T, z((k3r + k5r + kpr, c1), f32)], 0)
    col3 = jnp.concatenate([z((k1r, c3), f32), _pack(w_b2b).T,
                            z((k5r + kpr, c3), f32)], 0)
    col5 = jnp.concatenate([z((k1r + k3r, c5), f32), _pack(w_b3b).T,
                            z((kpr, c5), f32)], 0)
    colp = jnp.concatenate([z((k1r + k3r + k5r, cp), f32), _pack(w_b4).T], 0)
    wbig = jnp.concatenate([col1, col3, col5, colp], axis=1).astype(bf16)
    bbig = jnp.concatenate([b_b1, b_b2b, b_b3b, b_b4]).reshape(1, -1)

    kern = functools.partial(_kernel_v2, h, w, k3, k5,
                             c1, c3, c5, cp, cr3, cr5, cin, n)
    rowi = lambda t: (jnp.minimum(t, h - 1), 0, 0, 0)
    rowo = lambda t: (jnp.maximum(t - 2, 0), 0, 0, 0)
    rep2 = lambda t: (0, 0)
    out = pl.pallas_call(
        kern,
        out_shape=jax.ShapeDtypeStruct((h, w, n, ct), jnp.float32),
        grid=(h + 2,),
        in_specs=[
            pl.BlockSpec((1, w, cin, n), rowi),
            pl.BlockSpec((cin, cr3 + cr5), rep2),
            pl.BlockSpec((kt, ct), rep2),
            pl.BlockSpec((1, ct), rep2),
            pl.BlockSpec((cr3 + cr5, 1), rep2),
        ],
        out_specs=pl.BlockSpec((1, w, n, ct), rowo),
        scratch_shapes=[
            pltpu.VMEM((4, w, cin, n), bf16),
            pltpu.VMEM((8, w, cr3 + cr5, n), bf16),
        ],
        compiler_params=pltpu.CompilerParams(
            dimension_semantics=("arbitrary",),
            fuse_transposed_lhs_in_matmul=True,
            vmem_limit_bytes=48 * 1024 * 1024),
    )(xt, wredt, wbig, bbig, bredc)
    return jnp.transpose(out, (2, 3, 0, 1))


# speedup vs baseline: 1.6765x; 1.1383x over previous
import functools

import jax
import jax.numpy as jnp
from jax.experimental import pallas as pl
from jax.experimental.pallas import tpu as pltpu


def _pack(w):
    co, ci, k, _ = w.shape
    return jnp.transpose(w, (0, 2, 3, 1)).reshape(co, k * k * ci)


def _nc(taps, w_t, b, pref=jnp.float32):
    y = jnp.einsum('kn,kc->nc', taps, w_t, preferred_element_type=pref)
    return jnp.maximum(y + b[...], 0.0)


def _kernel_v2(h, w, k3, k5, c1, c3, c5, cp, cr3, cr5, cin, n,
               x_ref, wredt_ref, wbig_ref, bbig_ref, bredc_ref, o_ref,
               xring, yring):
    f32 = jnp.float32
    bf16 = jnp.bfloat16
    crt = cr3 + cr5
    t = pl.program_id(0)

    @pl.when(t == 0)
    def _init():
        xring[3] = jnp.zeros((w, cin, n), bf16)
        yring[6] = jnp.zeros((w, crt, n), bf16)
        yring[7] = jnp.zeros((w, crt, n), bf16)

    px = 2 if w % 2 == 0 else 1

    @pl.when(t < h)
    def _stage1():
        xring[t & 3] = x_ref[0]
        for col in range(0, w, px):
            xp = jnp.concatenate([x_ref[0, col + j] for j in range(px)],
                                 axis=-1)
            yr = jnp.einsum('ck,cn->kn', wredt_ref[...], xp,
                            preferred_element_type=f32)
            yr = jnp.maximum(yr + bredc_ref[...], 0.0).astype(bf16)
            for j in range(px):
                yring[t & 7, col + j] = yr[:, j * n:(j + 1) * n]

    @pl.when(t >= h)
    def _pad_rows():
        yring[t & 7] = jnp.zeros((w, crt, n), bf16)

    @pl.when(t >= 2)
    def _emit_row():
        r = t - 2
        z3 = jnp.zeros((cr3, n), bf16)
        z5 = jnp.zeros((cr5, n), bf16)

        rm_up = jnp.where(r >= 1, f32(1.0), f32(0.0)).astype(bf16)
        rm_dn = jnp.where(r <= h - 2, f32(1.0), f32(0.0)).astype(bf16)
        neg_up = jnp.where(r >= 1, f32(0.0), f32(-1e30)).astype(bf16)
        neg_dn = jnp.where(r <= h - 2, f32(0.0), f32(-1e30)).astype(bf16)

        hrows = []
        for d in (-3, -2, -1):
            xs = xring.at[(t + d) & 3]
            loads = [xs[col] for col in range(w)]
            hr = []
            for col in range(w):
                m = loads[col]
                if col > 0:
                    m = jnp.maximum(m, loads[col - 1])
                if col < w - 1:
                    m = jnp.maximum(m, loads[col + 1])
                hr.append(m)
            hrows.append(hr)
        hup, hmid, hdn = hrows

        yup = yring.at[(t - 3) & 7]
        ymid = yring.at[(t - 2) & 7]
        ydn = yring.at[(t - 1) & 7]
        y3rows = (yup, ymid, ydn)
        y5rows = tuple(yring.at[(t - 4 + i) & 7] for i in range(k5))

        def lhs_for(col):
            t3 = []
            for rw in y3rows:
                for dx in range(-(k3 // 2), k3 // 2 + 1):
                    cc = col + dx
                    t3.append(rw[cc, 0:cr3] if 0 <= cc < w else z3)
            t5 = []
            for rw in y5rows:
                for dx in range(-(k5 // 2), k5 // 2 + 1):
                    cc = col + dx
                    t5.append(rw[cc, cr3:crt] if 0 <= cc < w else z5)
            pooled = jnp.maximum(
                hmid[col],
                jnp.maximum(hup[col] * rm_up + neg_up,
                            hdn[col] * rm_dn + neg_dn))
            xp = xring[(t - 2) & 3, col]
            return jnp.concatenate([xp] + t3 + t5 + [pooled], axis=0)

        for col in range(0, w, px):
            lhs = jnp.concatenate([lhs_for(col + j) for j in range(px)],
                                  axis=-1)
            y = _nc(lhs, wbig_ref[...], bbig_ref)
            for j in range(px):
                o_ref[0, col + j] = y[j * n:(j + 1) * n]


@jax.jit
def kernel(x, w_b1, b_b1, w_b2a, b_b2a, w_b2b, b_b2b,
           w_b3a, b_b3a, w_b3b, b_b3b, w_b4, b_b4):
    n, cin, h, w = x.shape
    k3, k5 = w_b2b.shape[2], w_b3b.shape[2]
    c1, c3, c5, cp = w_b1.shape[0], w_b2b.shape[0], w_b3b.shape[0], w_b4.shape[0]
    cr3, cr5 = w_b2a.shape[0], w_b3a.shape[0]
    ct = c1 + c3 + c5 + cp

    bf16 = jnp.bfloat16
    xt = jnp.transpose(x, (2, 3, 1, 0)).astype(bf16)

    wredt = jnp.concatenate([_pack(w_b2a), _pack(w_b3a)], axis=0).T.astype(bf16)
    bredc = jnp.concatenate([b_b2a, b_b3a]).reshape(-1, 1)

    k1r, k3r, k5r, kpr = cin, k3 * k3 * cr3, k5 * k5 * cr5, cin
    kt = k1r + k3r + k5r + kpr
    z = jnp.zeros
    f32 = jnp.float32
    col1 = jnp.concatenate([_pack(w_b1).T, z((k3r + k5r + kpr, c1), f32)], 0)
    col3 = jnp.concatenate([z((k1r, c3), f32), _pack(w_b2b).T,
                            z((k5r + kpr, c3), f32)], 0)
    col5 = jnp.concatenate([z((k1r + k3r, c5), f32), _pack(w_b3b).T,
                            z((kpr, c5), f32)], 0)
    colp = jnp.concatenate([z((k1r + k3r + k5r, cp), f32), _pack(w_b4).T], 0)
    wbig = jnp.concatenate([col1, col3, col5, colp], axis=1).astype(bf16)
    bbig = jnp.concatenate([b_b1, b_b2b, b_b3b, b_b4]).reshape(1, -1)

    kern = functools.partial(_kernel_v2, h, w, k3, k5,
                             c1, c3, c5, cp, cr3, cr5, cin, n)
    rowi = lambda t: (jnp.minimum(t, h - 1), 0, 0, 0)
    rowo = lambda t: (jnp.maximum(t - 2, 0), 0, 0, 0)
    rep2 = lambda t: (0, 0)
    out = pl.pallas_call(
        kern,
        out_shape=jax.ShapeDtypeStruct((h, w, n, ct), jnp.float32),
        grid=(h + 2,),
        in_specs=[
            pl.BlockSpec((1, w, cin, n), rowi),
            pl.BlockSpec((cin, cr3 + cr5), rep2),
            pl.BlockSpec((kt, ct), rep2),
            pl.BlockSpec((1, ct), rep2),
            pl.BlockSpec((cr3 + cr5, 1), rep2),
        ],
        out_specs=pl.BlockSpec((1, w, n, ct), rowo),
        scratch_shapes=[
            pltpu.VMEM((4, w, cin, n), bf16),
            pltpu.VMEM((8, w, cr3 + cr5, n), bf16),
        ],
        compiler_params=pltpu.CompilerParams(
            dimension_semantics=("arbitrary",),
            fuse_transposed_lhs_in_matmul=True,
            allow_input_fusion=[True] + [False] * 4,
            vmem_limit_bytes=48 * 1024 * 1024),
    )(xt, wredt, wbig, bbig, bredc)
    return jnp.transpose(out, (2, 3, 0, 1))
